# baseline (speedup 1.0000x reference)
import functools

import jax
import jax.numpy as jnp
from jax.experimental import pallas as pl

N = 50000
E = 800000
N_TYPES = 37
NUM_BASIS = 8
R_MAX = 6.0
P = 6
MULT = 16
LATENT = 64
OUT_DIM = 64
SH_DIM = 9
N_GRAPHS = 32
N_RES = 6250

_LANES = 128
_EROWS = E // _LANES
_RBLK = 64
_NGEO = 18


def _silu(x):
    return x * jax.nn.sigmoid(x)


def _geom_kernel(p_ref, g_ref):
    vx = p_ref[3] - p_ref[0]
    vy = p_ref[4] - p_ref[1]
    vz = p_ref[5] - p_ref[2]
    r2 = vx * vx + vy * vy + vz * vz + 1e-12
    r = jnp.sqrt(r2)
    inv_r = 1.0 / r
    u = jnp.clip(r * (1.0 / R_MAX), 0.0, 1.0)
    u2 = u * u
    up = u2 * u2 * u2
    fcut = (1.0 - ((P + 1.0) * (P + 2.0) / 2.0) * up
            + P * (P + 2.0) * up * u
            - (P * (P + 1.0) / 2.0) * up * u2)
    scale = jnp.sqrt(2.0 / R_MAX) * fcut * inv_r
    for k in range(NUM_BASIS):
        g_ref[k] = jnp.sin(((k + 1) * jnp.pi / R_MAX) * r) * scale
    x = vx * inv_r
    y = vy * inv_r
    z = vz * inv_r
    c1 = jnp.sqrt(3.0)
    c2 = jnp.sqrt(15.0)
    g_ref[8] = jnp.ones_like(x)
    g_ref[9] = c1 * x
    g_ref[10] = c1 * y
    g_ref[11] = c1 * z
    g_ref[12] = c2 * x * y
    g_ref[13] = c2 * y * z
    g_ref[14] = (jnp.sqrt(5.0) / 2.0) * (3.0 * z * z - 1.0)
    g_ref[15] = c2 * x * z
    g_ref[16] = (c2 / 2.0) * (x * x - y * y)
    g_ref[17] = fcut


def _edge_geometry(pos_src, pos_dst):
    p = jnp.concatenate([pos_src.T, pos_dst.T], axis=0).reshape(6, _EROWS, _LANES)
    g = pl.pallas_call(
        _geom_kernel,
        grid=((_EROWS + _RBLK - 1) // _RBLK,),
        in_specs=[pl.BlockSpec((6, _RBLK, _LANES), lambda i: (0, i, 0))],
        out_specs=pl.BlockSpec((_NGEO, _RBLK, _LANES), lambda i: (0, i, 0)),
        out_shape=jax.ShapeDtypeStruct((_NGEO, _EROWS, _LANES), jnp.float32),
    )(p)
    g = g.reshape(_NGEO, E)
    edge_emb = g[0:NUM_BASIS].T
    sh = g[NUM_BASIS:NUM_BASIS + SH_DIM].T
    fcut = g[17]
    return edge_emb, sh, fcut


def kernel(pos, atom_type, edge_index, batch, residue_index,
           W_two, Wenv0, Wenv1, Wlat0, Wlat1, Wout):
    src = edge_index[0]
    dst = edge_index[1]
    pos_src = pos[src]
    pos_dst = pos[dst]
    edge_emb, sh, fcut = _edge_geometry(pos_src, pos_dst)

    onehot = jax.nn.one_hot(atom_type, N_TYPES, dtype=jnp.float32)
    two_in = jnp.concatenate([onehot[src], onehot[dst], edge_emb], axis=-1)
    latent = _silu(two_in @ W_two)
    avg_nn = jnp.asarray(E / N, dtype=jnp.float32)
    inv_sqrt_nn = 1.0 / jnp.sqrt(avg_nn)
    for Wenv, Wlat in ((Wenv0, Wlat0), (Wenv1, Wlat1)):
        env_w = (latent @ Wenv) * fcut[:, None]
        scalars = jnp.zeros((E, MULT), jnp.float32)
        for c in range(SH_DIM):
            shc = sh[:, c][:, None]
            env_c = jax.ops.segment_sum(env_w * shc, dst, num_segments=N) * inv_sqrt_nn
            scalars = scalars + env_c[dst] * shc
        latent_new = _silu(jnp.concatenate([latent, scalars], axis=-1) @ Wlat)
        latent = latent + latent_new
    edge_energy = (latent @ Wout) * fcut[:, None]
    node_energy = jax.ops.segment_sum(edge_energy, dst, num_segments=N)
    ones = jnp.ones((N, 1), jnp.float32)
    res_sum = jax.ops.segment_sum(node_energy, residue_index, num_segments=N_RES)
    res_cnt = jax.ops.segment_sum(ones, residue_index, num_segments=N_RES)
    residue_embedding = res_sum / jnp.clip(res_cnt, 1.0)
    g_sum = jax.ops.segment_sum(node_energy, batch, num_segments=N_GRAPHS)
    g_cnt = jax.ops.segment_sum(ones, batch, num_segments=N_GRAPHS)
    graph_embedding = g_sum / jnp.clip(g_cnt, 1.0)
    return (node_energy, residue_embedding, graph_embedding)


# speedup vs baseline: 2.0207x; 2.0207x over previous
import functools

import jax
import jax.numpy as jnp
from jax import lax
from jax.experimental import pallas as pl
from jax.experimental.pallas import tpu as pltpu
from jax.experimental.pallas import tpu_sc as plsc

N = 50000
E = 800000
N_TYPES = 37
NUM_BASIS = 8
R_MAX = 6.0
P = 6
MULT = 16
LATENT = 64
OUT_DIM = 64
SH_DIM = 9
N_GRAPHS = 32
N_RES = 6250

E2 = 819200
N2 = 51200
R2 = 6400
G2 = 48
CH = 3200
CROWS = CH // 128
EB = 2048
INV_SQRT_NN = 0.25

_MESH = plsc.VectorSubcoreMesh(core_axis_name="c", subcore_axis_name="s")


def _silu(x):
    return x * jax.nn.sigmoid(x)


@functools.partial(
    pl.kernel,
    out_type=jax.ShapeDtypeStruct((2, E2, 16), jnp.float32),
    mesh=_MESH,
    scratch_types=[
        pltpu.VMEM((CROWS, 128), jnp.int32),
        pltpu.VMEM((CH, 16), jnp.float32),
        pltpu.SemaphoreType.DMA,
    ],
    compiler_params=pltpu.CompilerParams(use_tc_tiling_on_sc=False),
)
def _sc_gather_edge_rows(tab_hbm, idx_hbm, out_hbm, idx_v, rows_v, sem):
    wid = lax.axis_index("s") * 2 + lax.axis_index("c")
    for s in range(2):
        def chunk(k, _, s=s):
            base = wid * 25600 + k * CH
            pltpu.sync_copy(idx_hbm.at[s].at[wid * 8 + k], idx_v)
            hs = []
            for j in range(CROWS):
                hs.append(pltpu.async_copy(
                    tab_hbm.at[idx_v.at[j]],
                    rows_v.at[pl.ds(j * 128, 128)], sem))
            for h in hs:
                h.wait()
            pltpu.sync_copy(rows_v, out_hbm.at[s].at[pl.ds(base, CH)])
            return 0
        lax.fori_loop(0, 25600 // CH, chunk, 0)


@functools.partial(
    pl.kernel,
    out_type=jax.ShapeDtypeStruct((SH_DIM, E2, MULT), jnp.float32),
    mesh=_MESH,
    scratch_types=[
        pltpu.VMEM_SHARED((N2, MULT), jnp.float32),
        pltpu.VMEM((CROWS, 128), jnp.int32),
        pltpu.VMEM((CH, MULT), jnp.float32),
        pltpu.SemaphoreType.DMA,
    ],
    compiler_params=pltpu.CompilerParams(use_tc_tiling_on_sc=False),
)
def _sc_env_round(vals_hbm, idx_hbm, out_hbm, acc, idx_v, vals_v, sem):
    cid = lax.axis_index("c")
    sid = lax.axis_index("s")

    def zfill(r, _):
        vals_v[r, :] = jnp.zeros((MULT,), jnp.float32)
        return 0

    for i in range(5):
        c = 2 * i
        cdyn = c + cid
        valid = cdyn <= 8

        @pl.when(valid)
        def _():
            lax.fori_loop(0, CH, zfill, 0)
            pltpu.sync_copy(vals_v, acc.at[pl.ds(sid * CH, CH)])
            plsc.subcore_barrier()

            def sc_chunk(k, _):
                base = sid * 51200 + k * CH
                pltpu.sync_copy(idx_hbm.at[sid * 16 + k], idx_v)
                pltpu.sync_copy(vals_hbm.at[cdyn].at[pl.ds(base, CH)], vals_v)
                for j in range(CROWS):
                    pltpu.sync_copy(vals_v.at[pl.ds(j * 128, 128)],
                                    acc.at[idx_v.at[j]], add=True)
                return 0
            lax.fori_loop(0, 16, sc_chunk, 0)
            plsc.subcore_barrier()

            def g_chunk(k, _):
                base = sid * 51200 + k * CH
                pltpu.sync_copy(idx_hbm.at[sid * 16 + k], idx_v)
                hs = []
                for j in range(CROWS):
                    hs.append(pltpu.async_copy(
                        acc.at[idx_v.at[j]],
                        vals_v.at[pl.ds(j * 128, 128)], sem))
                for h in hs:
                    h.wait()
                pltpu.sync_copy(vals_v, out_hbm.at[cdyn].at[pl.ds(base, CH)])
                return 0
            lax.fori_loop(0, 16, g_chunk, 0)
            plsc.subcore_barrier()


@functools.partial(
    pl.kernel,
    out_type=jax.ShapeDtypeStruct((4, N2, MULT), jnp.float32),
    mesh=_MESH,
    scratch_types=[
        pltpu.VMEM_SHARED((N2, MULT), jnp.float32),
        pltpu.VMEM((CROWS, 128), jnp.int32),
        pltpu.VMEM((CH, MULT), jnp.float32),
        pltpu.SemaphoreType.DMA,
    ],
    compiler_params=pltpu.CompilerParams(use_tc_tiling_on_sc=False),
)
def _sc_node_energy(vals_hbm, idx_hbm, out_hbm, acc, idx_v, vals_v, sem):
    cid = lax.axis_index("c")
    sid = lax.axis_index("s")

    def zfill(r, _):
        vals_v[r, :] = jnp.zeros((MULT,), jnp.float32)
        return 0

    for i in range(2):
        pdyn = 2 * i + cid
        lax.fori_loop(0, CH, zfill, 0)
        pltpu.sync_copy(vals_v, acc.at[pl.ds(sid * CH, CH)])
        plsc.subcore_barrier()

        def sc_chunk(k, _):
            base = sid * 51200 + k * CH
            pltpu.sync_copy(idx_hbm.at[sid * 16 + k], idx_v)
            pltpu.sync_copy(vals_hbm.at[pdyn].at[pl.ds(base, CH)], vals_v)
            for j in range(CROWS):
                pltpu.sync_copy(vals_v.at[pl.ds(j * 128, 128)],
                                acc.at[idx_v.at[j]], add=True)
            return 0
        lax.fori_loop(0, 16, sc_chunk, 0)
        plsc.subcore_barrier()
        pltpu.sync_copy(acc.at[pl.ds(sid * CH, CH)],
                        out_hbm.at[pdyn].at[pl.ds(sid * CH, CH)])
        plsc.subcore_barrier()


@functools.partial(
    pl.kernel,
    out_type=[
        jax.ShapeDtypeStruct((4, R2, MULT), jnp.float32),
        jax.ShapeDtypeStruct((R2, MULT), jnp.float32),
        jax.ShapeDtypeStruct((4, G2, MULT), jnp.float32),
        jax.ShapeDtypeStruct((G2, MULT), jnp.float32),
    ],
    mesh=_MESH,
    scratch_types=[
        pltpu.VMEM_SHARED((R2, MULT), jnp.float32),
        pltpu.VMEM_SHARED((G2, MULT), jnp.float32),
        pltpu.VMEM((CROWS, 128), jnp.int32),
        pltpu.VMEM((CH, MULT), jnp.float32),
        pltpu.VMEM((CH, MULT), jnp.float32),
        pltpu.SemaphoreType.DMA,
    ],
    compiler_params=pltpu.CompilerParams(use_tc_tiling_on_sc=False),
)
def _sc_segment_sums(ne_hbm, rid_hbm, gid_hbm, rsum_hbm, rcnt_hbm,
                     gsum_hbm, gcnt_hbm, racc, gacc, idx_v, vals_v, ones_v,
                     sem):
    cid = lax.axis_index("c")
    sid = lax.axis_index("s")

    def ofill(r, _):
        ones_v[r, :] = jnp.ones((MULT,), jnp.float32)
        return 0
    lax.fori_loop(0, CH, ofill, 0)

    def scatter_pass(idx_hbm_ref, acc_ref, src_hbm, use_ones):
        pltpu.sync_copy(idx_hbm_ref, idx_v)
        if not use_ones:
            pltpu.sync_copy(src_hbm, vals_v)
        srcbuf = ones_v if use_ones else vals_v
        for j in range(CROWS):
            pltpu.sync_copy(srcbuf.at[pl.ds(j * 128, 128)],
                            acc_ref.at[idx_v.at[j]], add=True)

    for i in range(2):
        pdyn = 2 * i + cid
        def zfill(r, _):
            vals_v[r, :] = jnp.zeros((MULT,), jnp.float32)
            return 0
        lax.fori_loop(0, 400, zfill, 0)
        pltpu.sync_copy(vals_v.at[pl.ds(0, 400)],
                        racc.at[pl.ds(sid * 400, 400)])

        @pl.when(sid == 0)
        def _():
            pltpu.sync_copy(vals_v.at[pl.ds(0, G2)], gacc)
        plsc.subcore_barrier()

        scatter_pass(rid_hbm.at[sid], racc,
                     ne_hbm.at[pdyn].at[pl.ds(sid * CH, CH)], False)
        scatter_pass(gid_hbm.at[sid], gacc,
                     ne_hbm.at[pdyn].at[pl.ds(sid * CH, CH)], False)
        plsc.subcore_barrier()
        pltpu.sync_copy(racc.at[pl.ds(sid * 400, 400)],
                        rsum_hbm.at[pdyn].at[pl.ds(sid * 400, 400)])

        @pl.when(sid == 0)
        def _():
            pltpu.sync_copy(gacc, gsum_hbm.at[pdyn])
        plsc.subcore_barrier()

    @pl.when(cid == 0)
    def _():
        def zfill(r, _):
            vals_v[r, :] = jnp.zeros((MULT,), jnp.float32)
            return 0
        lax.fori_loop(0, 400, zfill, 0)
        pltpu.sync_copy(vals_v.at[pl.ds(0, 400)],
                        racc.at[pl.ds(sid * 400, 400)])

        @pl.when(sid == 0)
        def _():
            pltpu.sync_copy(vals_v.at[pl.ds(0, G2)], gacc)
        plsc.subcore_barrier()
        scatter_pass(rid_hbm.at[sid], racc, None, True)
        scatter_pass(gid_hbm.at[sid], gacc, None, True)
        plsc.subcore_barrier()
        pltpu.sync_copy(racc.at[pl.ds(sid * 400, 400)],
                        rcnt_hbm.at[pl.ds(sid * 400, 400)])

        @pl.when(sid == 0)
        def _():
            pltpu.sync_copy(gacc, gcnt_hbm)


def _geom_feats(tsT, tdT):
    vx = tdT[0:1] - tsT[0:1]
    vy = tdT[1:2] - tsT[1:2]
    vz = tdT[2:3] - tsT[2:3]
    r2 = vx * vx + vy * vy + vz * vz + 1e-12
    r = jnp.sqrt(r2)
    inv_r = 1.0 / r
    u = jnp.clip(r * (1.0 / R_MAX), 0.0, 1.0)
    u2 = u * u
    up = u2 * u2 * u2
    fcut = (1.0 - ((P + 1.0) * (P + 2.0) / 2.0) * up
            + P * (P + 2.0) * up * u
            - (P * (P + 1.0) / 2.0) * up * u2)
    scale = jnp.sqrt(2.0 / R_MAX) * fcut * inv_r
    emb = jnp.concatenate(
        [jnp.sin(((k + 1) * jnp.pi / R_MAX) * r) * scale
         for k in range(NUM_BASIS)], axis=0)
    x = vx * inv_r
    y = vy * inv_r
    z = vz * inv_r
    c1 = jnp.sqrt(3.0)
    c2 = jnp.sqrt(15.0)
    sh = jnp.concatenate([
        jnp.ones_like(x), c1 * x, c1 * y, c1 * z,
        c2 * x * y, c2 * y * z,
        (jnp.sqrt(5.0) / 2.0) * (3.0 * z * z - 1.0),
        c2 * x * z, (c2 / 2.0) * (x * x - y * y)], axis=0)
    return fcut, emb, sh


def _onehotT(trow, B):
    io = lax.broadcasted_iota(jnp.int32, (N_TYPES, B), 0).astype(jnp.float32)
    return jnp.where(io == trow, 1.0, 0.0)


def _latent0T(tsT, tdT, emb, W_twoT_ref, B):
    ohs = _onehotT(tsT[3:4], B)
    ohd = _onehotT(tdT[3:4], B)
    two_inT = jnp.concatenate([ohs, ohd, emb], axis=0)
    return _silu(jnp.dot(W_twoT_ref[...], two_inT,
                         preferred_element_type=jnp.float32))


def _t2_kernel(ts_ref, td_ref, W_twoT_ref, WenvT_ref, ev_ref):
    i = pl.program_id(0)
    B = EB
    tsT = ts_ref[...].T
    tdT = td_ref[...].T
    fcut, emb, sh = _geom_feats(tsT, tdT)
    latT = _latent0T(tsT, tdT, emb, W_twoT_ref, B)
    gidx = i * B + lax.broadcasted_iota(jnp.int32, (1, B), 1)
    mask = jnp.where(gidx < E, 1.0, 0.0)
    env_wT = jnp.dot(WenvT_ref[...], latT,
                     preferred_element_type=jnp.float32) * fcut * mask
    for c in range(SH_DIM):
        ev_ref[c, :, :] = (env_wT * sh[c:c + 1]).T


def _t4_kernel(ts_ref, td_ref, eg_ref, W_twoT_ref, WlatT_ref, WenvT_ref,
               ev_ref, lat_ref):
    i = pl.program_id(0)
    B = EB
    tsT = ts_ref[...].T
    tdT = td_ref[...].T
    fcut, emb, sh = _geom_feats(tsT, tdT)
    latT = _latent0T(tsT, tdT, emb, W_twoT_ref, B)
    scal = jnp.zeros((MULT, B), jnp.float32)
    for c in range(SH_DIM):
        scal = scal + eg_ref[c, :, :].T * sh[c:c + 1]
    scal = scal * INV_SQRT_NN
    catT = jnp.concatenate([latT, scal], axis=0)
    lat1T = latT + _silu(jnp.dot(WlatT_ref[...], catT,
                                 preferred_element_type=jnp.float32))
    lat_ref[...] = lat1T
    gidx = i * B + lax.broadcasted_iota(jnp.int32, (1, B), 1)
    mask = jnp.where(gidx < E, 1.0, 0.0)
    env_wT = jnp.dot(WenvT_ref[...], lat1T,
                     preferred_element_type=jnp.float32) * fcut * mask
    for c in range(SH_DIM):
        ev_ref[c, :, :] = (env_wT * sh[c:c + 1]).T


def _t6_kernel(ts_ref, td_ref, lat_ref, eg_ref, WlatT_ref, WoutT_ref, ee_ref):
    i = pl.program_id(0)
    B = EB
    tsT = ts_ref[...].T
    tdT = td_ref[...].T
    fcut, _, sh = _geom_feats(tsT, tdT)
    lat1T = lat_ref[...]
    scal = jnp.zeros((MULT, B), jnp.float32)
    for c in range(SH_DIM):
        scal = scal + eg_ref[c, :, :].T * sh[c:c + 1]
    scal = scal * INV_SQRT_NN
    catT = jnp.concatenate([lat1T, scal], axis=0)
    lat2T = lat1T + _silu(jnp.dot(WlatT_ref[...], catT,
                                  preferred_element_type=jnp.float32))
    gidx = i * B + lax.broadcasted_iota(jnp.int32, (1, B), 1)
    mask = jnp.where(gidx < E, 1.0, 0.0)
    eeT = jnp.dot(WoutT_ref[...], lat2T,
                  preferred_element_type=jnp.float32) * fcut * mask
    for p in range(4):
        ee_ref[p, :, :] = eeT[p * MULT:(p + 1) * MULT].T


def _t9_kernel(ne4_ref, out_ref):
    out_ref[...] = jnp.concatenate(
        [ne4_ref[p, :, :] for p in range(4)], axis=-1)


def _t10_kernel(rsum_ref, rcnt_ref, gsum_ref, gcnt_ref, res_ref, g_ref):
    rs = jnp.concatenate([rsum_ref[p, :, :] for p in range(4)], axis=-1)
    rc = jnp.clip(rcnt_ref[:, 0:1], 1.0, None)
    res_ref[...] = (rs / rc)[:N_RES]
    gs = jnp.concatenate([gsum_ref[p, :, :] for p in range(4)], axis=-1)
    gc = jnp.clip(gcnt_ref[:, 0:1], 1.0, None)
    g_ref[...] = (gs / gc)[:N_GRAPHS]


def kernel(pos, atom_type, edge_index, batch, residue_index,
           W_two, Wenv0, Wenv1, Wlat0, Wlat1, Wout):
    f32 = jnp.float32
    src = edge_index[0].astype(jnp.int32)
    dst = edge_index[1].astype(jnp.int32)
    epad = jnp.arange(E2 - E, dtype=jnp.int32) % N
    srcp = jnp.concatenate([src, epad])
    dstp = jnp.concatenate([dst, epad])
    idx2 = jnp.stack([srcp, dstp]).reshape(2, E2 // CH, CROWS, 128)
    dst2 = dstp.reshape(E2 // CH, CROWS, 128)

    tab = jnp.concatenate(
        [pos.astype(f32), atom_type.astype(f32)[:, None],
         jnp.zeros((N, 12), f32)], axis=1)

    npad = jnp.arange(N2 - N, dtype=jnp.int32)
    rid = jnp.concatenate([residue_index.astype(jnp.int32),
                           N_RES + (npad % (R2 - N_RES))])
    gid = jnp.concatenate([batch.astype(jnp.int32),
                           N_GRAPHS + (npad % (G2 - N_GRAPHS))])
    rid2 = rid.reshape(N2 // CH, CROWS, 128)
    gid2 = gid.reshape(N2 // CH, CROWS, 128)

    W_twoT = W_two.T
    Wenv0T = Wenv0.T
    Wenv1T = Wenv1.T
    Wlat0T = Wlat0.T
    Wlat1T = Wlat1.T
    WoutT = Wout.T

    tsd = _sc_gather_edge_rows(tab, idx2)
    ts, td = tsd[0], tsd[1]

    grid = (E2 // EB,)
    ebs = pl.BlockSpec((EB, 16), lambda i: (i, 0))
    evs = pl.BlockSpec((SH_DIM, EB, MULT), lambda i: (0, i, 0))
    lbs = pl.BlockSpec((LATENT, EB), lambda i: (0, i))
    wspec = lambda s: pl.BlockSpec(s, lambda i: tuple(0 for _ in s))

    ev0 = pl.pallas_call(
        _t2_kernel, grid=grid,
        in_specs=[ebs, ebs, wspec((LATENT, 82)), wspec((MULT, LATENT))],
        out_specs=evs,
        out_shape=jax.ShapeDtypeStruct((SH_DIM, E2, MULT), f32),
    )(ts, td, W_twoT, Wenv0T)

    eg0 = _sc_env_round(ev0, dst2)

    ev1, lat1 = pl.pallas_call(
        _t4_kernel, grid=grid,
        in_specs=[ebs, ebs, evs, wspec((LATENT, 82)),
                  wspec((LATENT, LATENT + MULT)), wspec((MULT, LATENT))],
        out_specs=[evs, lbs],
        out_shape=[jax.ShapeDtypeStruct((SH_DIM, E2, MULT), f32),
                   jax.ShapeDtypeStruct((LATENT, E2), f32)],
    )(ts, td, eg0, W_twoT, Wlat0T, Wenv1T)

    eg1 = _sc_env_round(ev1, dst2)

    ee4 = pl.pallas_call(
        _t6_kernel, grid=grid,
        in_specs=[ebs, ebs, lbs, evs,
                  wspec((LATENT, LATENT + MULT)), wspec((OUT_DIM, LATENT))],
        out_specs=pl.BlockSpec((4, EB, MULT), lambda i: (0, i, 0)),
        out_shape=jax.ShapeDtypeStruct((4, E2, MULT), f32),
    )(ts, td, lat1, eg1, Wlat1T, WoutT)

    ne4 = _sc_node_energy(ee4, dst2)

    rsum, rcnt, gsum, gcnt = _sc_segment_sums(ne4, rid2, gid2)

    NB = 1000
    node_energy = pl.pallas_call(
        _t9_kernel, grid=(N // NB,),
        in_specs=[pl.BlockSpec((4, NB, MULT), lambda i: (0, i, 0))],
        out_specs=pl.BlockSpec((NB, OUT_DIM), lambda i: (i, 0)),
        out_shape=jax.ShapeDtypeStruct((N, OUT_DIM), f32),
    )(ne4)

    residue_embedding, graph_embedding = pl.pallas_call(
        _t10_kernel,
        in_specs=[pl.BlockSpec((4, R2, MULT), lambda: (0, 0, 0)),
                  pl.BlockSpec((R2, MULT), lambda: (0, 0)),
                  pl.BlockSpec((4, G2, MULT), lambda: (0, 0, 0)),
                  pl.BlockSpec((G2, MULT), lambda: (0, 0))],
        out_specs=[pl.BlockSpec((N_RES, OUT_DIM), lambda: (0, 0)),
                   pl.BlockSpec((N_GRAPHS, OUT_DIM), lambda: (0, 0))],
        out_shape=[jax.ShapeDtypeStruct((N_RES, OUT_DIM), f32),
                   jax.ShapeDtypeStruct((N_GRAPHS, OUT_DIM), f32)],
    )(rsum, rcnt, gsum, gcnt)

    return (node_energy, residue_embedding, graph_embedding)


# speedup vs baseline: 2.8141x; 1.3926x over previous
import functools

import jax
import jax.numpy as jnp
from jax import lax
from jax.experimental import pallas as pl
from jax.experimental.pallas import tpu as pltpu
from jax.experimental.pallas import tpu_sc as plsc

N = 50000
E = 800000
N_TYPES = 37
NUM_BASIS = 8
R_MAX = 6.0
P = 6
MULT = 16
LATENT = 64
OUT_DIM = 64
SH_DIM = 9
N_GRAPHS = 32
N_RES = 6250

E2 = 819200
ER = E2 * 16 // 128
N2 = 51200
R2 = 6400
G2 = 48
CH = 3200
CROWS = CH // 128
NCH = 640
EB = 2048
BR = EB * 16 // 128
EG = EB // 8
INV_SQRT_NN = 0.25

_MESH = plsc.VectorSubcoreMesh(core_axis_name="c", subcore_axis_name="s")
_SC_PARAMS = pltpu.CompilerParams(use_tc_tiling_on_sc=False)


def _silu(x):
    return x * jax.nn.sigmoid(x)


@functools.partial(
    pl.kernel,
    out_type=jax.ShapeDtypeStruct((2, E2, 16), jnp.float32),
    mesh=_MESH,
    scratch_types=[
        pltpu.VMEM((CROWS, 128), jnp.int32),
        pltpu.VMEM((CH, 16), jnp.float32),
        pltpu.SemaphoreType.DMA,
    ],
    compiler_params=_SC_PARAMS,
)
def _sc_gather_edge_rows(tab_hbm, idx_hbm, out_hbm, idx_v, rows_v, sem):
    wid = lax.axis_index("s") * 2 + lax.axis_index("c")
    for s in range(2):
        def chunk(k, _, s=s):
            base = wid * 25600 + k * CH
            pltpu.sync_copy(idx_hbm.at[s].at[wid * 8 + k], idx_v)
            hs = []
            for j in range(CROWS):
                hs.append(pltpu.async_copy(
                    tab_hbm.at[idx_v.at[j]],
                    rows_v.at[pl.ds(j * 128, 128)], sem))
            for h in hs:
                h.wait()
            pltpu.sync_copy(rows_v, out_hbm.at[s].at[pl.ds(base, CH)])
            return 0
        lax.fori_loop(0, 25600 // CH, chunk, 0)


@functools.partial(
    pl.kernel,
    out_type=jax.ShapeDtypeStruct((SH_DIM, E2, MULT), jnp.float32),
    mesh=_MESH,
    scratch_types=[
        pltpu.VMEM_SHARED((N2, MULT), jnp.float32),
        pltpu.VMEM((CROWS, 128), jnp.int32),
        pltpu.VMEM((CH, MULT), jnp.float32),
        pltpu.SemaphoreType.DMA,
    ],
    compiler_params=_SC_PARAMS,
)
def _sc_env_round(vals_hbm, idx_hbm, out_hbm, acc, idx_v, vals_v, sem):
    cid = lax.axis_index("c")
    sid = lax.axis_index("s")

    def zfill(r, _):
        vals_v[r, :] = jnp.zeros((MULT,), jnp.float32)
        return 0

    for i in range(5):
        cdyn = 2 * i + cid
        valid = cdyn <= 8

        @pl.when(valid)
        def _():
            lax.fori_loop(0, CH, zfill, 0)
            pltpu.sync_copy(vals_v, acc.at[pl.ds(sid * CH, CH)])
            plsc.subcore_barrier()

            def sc_chunk(k, _):
                base = sid * 51200 + k * CH
                pltpu.sync_copy(idx_hbm.at[sid * 16 + k], idx_v)
                pltpu.sync_copy(vals_hbm.at[cdyn].at[pl.ds(base, CH)], vals_v)
                for j in range(CROWS):
                    pltpu.sync_copy(vals_v.at[pl.ds(j * 128, 128)],
                                    acc.at[idx_v.at[j]], add=True)
                return 0
            lax.fori_loop(0, 16, sc_chunk, 0)
            plsc.subcore_barrier()

            def g_chunk(k, _):
                base = sid * 51200 + k * CH
                pltpu.sync_copy(idx_hbm.at[sid * 16 + k], idx_v)
                hs = []
                for j in range(CROWS):
                    hs.append(pltpu.async_copy(
                        acc.at[idx_v.at[j]],
                        vals_v.at[pl.ds(j * 128, 128)], sem))
                for h in hs:
                    h.wait()
                pltpu.sync_copy(vals_v, out_hbm.at[cdyn].at[pl.ds(base, CH)])
                return 0
            lax.fori_loop(0, 16, g_chunk, 0)
            plsc.subcore_barrier()


@functools.partial(
    pl.kernel,
    out_type=jax.ShapeDtypeStruct((N2, OUT_DIM), jnp.float32),
    mesh=_MESH,
    scratch_types=[
        pltpu.VMEM_SHARED((N2, MULT), jnp.float32),
        pltpu.VMEM((CROWS, 128), jnp.int32),
        pltpu.VMEM((CH, MULT), jnp.float32),
        pltpu.SemaphoreType.DMA,
    ],
    compiler_params=_SC_PARAMS,
)
def _sc_node_energy(vals_hbm, idx_hbm, out_hbm, acc, idx_v, vals_v, sem):
    cid = lax.axis_index("c")
    sid = lax.axis_index("s")

    def zfill(r, _):
        vals_v[r, :] = jnp.zeros((MULT,), jnp.float32)
        return 0

    for i in range(2):
        pdyn = 2 * i + cid
        lax.fori_loop(0, CH, zfill, 0)
        pltpu.sync_copy(vals_v, acc.at[pl.ds(sid * CH, CH)])
        plsc.subcore_barrier()

        def sc_chunk(k, _):
            base = sid * 51200 + k * CH
            pltpu.sync_copy(idx_hbm.at[sid * 16 + k], idx_v)
            pltpu.sync_copy(vals_hbm.at[pdyn].at[pl.ds(base, CH)], vals_v)
            for j in range(CROWS):
                pltpu.sync_copy(vals_v.at[pl.ds(j * 128, 128)],
                                acc.at[idx_v.at[j]], add=True)
            return 0
        lax.fori_loop(0, 16, sc_chunk, 0)
        plsc.subcore_barrier()
        pltpu.sync_copy(acc.at[pl.ds(sid * CH, CH)], vals_v)
        pltpu.sync_copy(vals_v,
                        out_hbm.at[pl.ds(sid * CH, CH),
                                   pl.ds(pdyn * MULT, MULT)])
        plsc.subcore_barrier()


@functools.partial(
    pl.kernel,
    out_type=[
        jax.ShapeDtypeStruct((R2, OUT_DIM), jnp.float32),
        jax.ShapeDtypeStruct((R2, MULT), jnp.float32),
        jax.ShapeDtypeStruct((G2, OUT_DIM), jnp.float32),
        jax.ShapeDtypeStruct((G2, MULT), jnp.float32),
    ],
    mesh=_MESH,
    scratch_types=[
        pltpu.VMEM_SHARED((R2, OUT_DIM), jnp.float32),
        pltpu.VMEM_SHARED((G2, OUT_DIM), jnp.float32),
        pltpu.VMEM_SHARED((R2, MULT), jnp.float32),
        pltpu.VMEM_SHARED((G2, MULT), jnp.float32),
        pltpu.VMEM((5, 128), jnp.int32),
        pltpu.VMEM((NCH, OUT_DIM), jnp.float32),
        pltpu.VMEM((NCH, MULT), jnp.float32),
        pltpu.SemaphoreType.DMA,
    ],
    compiler_params=_SC_PARAMS,
)
def _sc_segment_sums(ne_hbm, rid_hbm, gid_hbm, rsum_hbm, rcnt_hbm,
                     gsum_hbm, gcnt_hbm, racc, gacc, rcacc, gcacc,
                     idx_v, vals_v, ones_v, sem):
    cid = lax.axis_index("c")
    sid = lax.axis_index("s")

    def zfill64(r, _):
        vals_v[r, :] = jnp.zeros((OUT_DIM,), jnp.float32)
        return 0

    def zfill16(r, _):
        ones_v[r, :] = jnp.zeros((MULT,), jnp.float32)
        return 0

    def ofill(r, _):
        ones_v[r, :] = jnp.ones((MULT,), jnp.float32)
        return 0

    lax.fori_loop(0, NCH, zfill64, 0)
    lax.fori_loop(0, NCH, zfill16, 0)

    @pl.when(cid == 0)
    def _():
        pltpu.sync_copy(vals_v.at[pl.ds(0, 400)],
                        racc.at[pl.ds(sid * 400, 400)])
        pltpu.sync_copy(ones_v.at[pl.ds(0, 400)],
                        rcacc.at[pl.ds(sid * 400, 400)])

    @pl.when(jnp.logical_and(cid == 1, sid == 0))
    def _():
        pltpu.sync_copy(vals_v.at[pl.ds(0, G2)], gacc)
        pltpu.sync_copy(ones_v.at[pl.ds(0, G2)], gcacc)

    lax.fori_loop(0, NCH, ofill, 0)
    plsc.subcore_barrier()

    def passes(idx_hbm_ref, acc_ref, cacc_ref):
        def n_chunk(k, _):
            g = sid * 5 + k
            pltpu.sync_copy(idx_hbm_ref.at[g], idx_v)
            pltpu.sync_copy(ne_hbm.at[pl.ds(g * NCH, NCH)], vals_v)
            for j in range(5):
                pltpu.sync_copy(vals_v.at[pl.ds(j * 128, 128)],
                                acc_ref.at[idx_v.at[j]], add=True)
                pltpu.sync_copy(ones_v.at[pl.ds(j * 128, 128)],
                                cacc_ref.at[idx_v.at[j]], add=True)
            return 0
        lax.fori_loop(0, 5, n_chunk, 0)

    @pl.when(cid == 0)
    def _():
        passes(rid_hbm, racc, rcacc)

    @pl.when(cid == 1)
    def _():
        passes(gid_hbm, gacc, gcacc)
    plsc.subcore_barrier()

    @pl.when(cid == 0)
    def _():
        pltpu.sync_copy(racc.at[pl.ds(sid * 400, 400)],
                        rsum_hbm.at[pl.ds(sid * 400, 400)])
        pltpu.sync_copy(rcacc.at[pl.ds(sid * 400, 400)],
                        rcnt_hbm.at[pl.ds(sid * 400, 400)])

    @pl.when(jnp.logical_and(cid == 1, sid == 0))
    def _():
        pltpu.sync_copy(gacc, gsum_hbm)
        pltpu.sync_copy(gcacc, gcnt_hbm)


def _load_T(block):
    return jnp.concatenate(
        [block[:, MULT * q:MULT * (q + 1)].T for q in range(8)], axis=1)


def _store_T(valT):
    return jnp.concatenate(
        [valT[:, EG * q:EG * (q + 1)].T for q in range(8)], axis=1)


def _geom_feats(tsT, tdT):
    vx = tdT[0:1] - tsT[0:1]
    vy = tdT[1:2] - tsT[1:2]
    vz = tdT[2:3] - tsT[2:3]
    r2 = vx * vx + vy * vy + vz * vz + 1e-12
    r = jnp.sqrt(r2)
    inv_r = 1.0 / r
    u = jnp.clip(r * (1.0 / R_MAX), 0.0, 1.0)
    u2 = u * u
    up = u2 * u2 * u2
    fcut = (1.0 - ((P + 1.0) * (P + 2.0) / 2.0) * up
            + P * (P + 2.0) * up * u
            - (P * (P + 1.0) / 2.0) * up * u2)
    scale = jnp.sqrt(2.0 / R_MAX) * fcut * inv_r
    emb = jnp.concatenate(
        [jnp.sin(((k + 1) * jnp.pi / R_MAX) * r) * scale
         for k in range(NUM_BASIS)], axis=0)
    x = vx * inv_r
    y = vy * inv_r
    z = vz * inv_r
    c1 = jnp.sqrt(3.0)
    c2 = jnp.sqrt(15.0)
    sh = jnp.concatenate([
        jnp.ones_like(x), c1 * x, c1 * y, c1 * z,
        c2 * x * y, c2 * y * z,
        (jnp.sqrt(5.0) / 2.0) * (3.0 * z * z - 1.0),
        c2 * x * z, (c2 / 2.0) * (x * x - y * y)], axis=0)
    return fcut, emb, sh


def _onehotT(trow, B):
    io = lax.broadcasted_iota(jnp.int32, (N_TYPES, B), 0).astype(jnp.float32)
    return jnp.where(io == trow, 1.0, 0.0)


def _latent0T(tsT, tdT, emb, W_twoT_ref, B):
    ohs = _onehotT(tsT[3:4], B)
    ohd = _onehotT(tdT[3:4], B)
    two_inT = jnp.concatenate([ohs, ohd, emb], axis=0)
    return _silu(jnp.dot(W_twoT_ref[...], two_inT,
                         preferred_element_type=jnp.float32))


def _edge_mask(i):
    l = lax.broadcasted_iota(jnp.int32, (1, EB), 1)
    e = i * EB + 8 * (l % EG) + l // EG
    return jnp.where(e < E, 1.0, 0.0)


def _t2_kernel(ts_ref, td_ref, W_twoT_ref, WenvT_ref, ev_ref):
    i = pl.program_id(0)
    tsT = _load_T(ts_ref[...])
    tdT = _load_T(td_ref[...])
    fcut, emb, sh = _geom_feats(tsT, tdT)
    latT = _latent0T(tsT, tdT, emb, W_twoT_ref, EB)
    env_wT = jnp.dot(WenvT_ref[...], latT,
                     preferred_element_type=jnp.float32) * fcut * _edge_mask(i)
    for c in range(SH_DIM):
        ev_ref[c, :, :] = _store_T(env_wT * sh[c:c + 1])


def _t4_kernel(ts_ref, td_ref, eg_ref, W_twoT_ref, WlatT_ref, WenvT_ref,
               ev_ref, lat_ref):
    i = pl.program_id(0)
    tsT = _load_T(ts_ref[...])
    tdT = _load_T(td_ref[...])
    fcut, emb, sh = _geom_feats(tsT, tdT)
    latT = _latent0T(tsT, tdT, emb, W_twoT_ref, EB)
    scal = jnp.zeros((MULT, EB), jnp.float32)
    for c in range(SH_DIM):
        scal = scal + _load_T(eg_ref[c, :, :]) * sh[c:c + 1]
    scal = scal * INV_SQRT_NN
    catT = jnp.concatenate([latT, scal], axis=0)
    lat1T = latT + _silu(jnp.dot(WlatT_ref[...], catT,
                                 preferred_element_type=jnp.float32))
    lat_ref[...] = lat1T
    env_wT = jnp.dot(WenvT_ref[...], lat1T,
                     preferred_element_type=jnp.float32) * fcut * _edge_mask(i)
    for c in range(SH_DIM):
        ev_ref[c, :, :] = _store_T(env_wT * sh[c:c + 1])


def _t6_kernel(ts_ref, td_ref, lat_ref, eg_ref, WlatT_ref, WoutT_ref, ee_ref):
    i = pl.program_id(0)
    tsT = _load_T(ts_ref[...])
    tdT = _load_T(td_ref[...])
    fcut, _, sh = _geom_feats(tsT, tdT)
    lat1T = lat_ref[...]
    scal = jnp.zeros((MULT, EB), jnp.float32)
    for c in range(SH_DIM):
        scal = scal + _load_T(eg_ref[c, :, :]) * sh[c:c + 1]
    scal = scal * INV_SQRT_NN
    catT = jnp.concatenate([lat1T, scal], axis=0)
    lat2T = lat1T + _silu(jnp.dot(WlatT_ref[...], catT,
                                  preferred_element_type=jnp.float32))
    eeT = jnp.dot(WoutT_ref[...], lat2T,
                  preferred_element_type=jnp.float32) * fcut * _edge_mask(i)
    for p in range(4):
        ee_ref[p, :, :] = _store_T(eeT[p * MULT:(p + 1) * MULT])


def _t10_kernel(rsum_ref, rcnt_ref, gsum_ref, gcnt_ref, res_ref, g_ref):
    rc = jnp.clip(rcnt_ref[:, 0:1], 1.0, None)
    res_ref[...] = (rsum_ref[...] / rc)[:N_RES]
    gc = jnp.clip(gcnt_ref[:, 0:1], 1.0, None)
    g_ref[...] = (gsum_ref[...] / gc)[:N_GRAPHS]


def kernel(pos, atom_type, edge_index, batch, residue_index,
           W_two, Wenv0, Wenv1, Wlat0, Wlat1, Wout):
    f32 = jnp.float32
    src = edge_index[0].astype(jnp.int32)
    dst = edge_index[1].astype(jnp.int32)
    epad = jnp.arange(E2 - E, dtype=jnp.int32) % N
    srcp = jnp.concatenate([src, epad])
    dstp = jnp.concatenate([dst, epad])
    idx2 = jnp.stack([srcp, dstp]).reshape(2, E2 // CH, CROWS, 128)
    dst2 = dstp.reshape(E2 // CH, CROWS, 128)

    tab = jnp.concatenate(
        [pos.astype(f32), atom_type.astype(f32)[:, None],
         jnp.zeros((N, 12), f32)], axis=1)

    npad = jnp.arange(N2 - N, dtype=jnp.int32)
    rid = jnp.concatenate([residue_index.astype(jnp.int32),
                           N_RES + (npad % (R2 - N_RES))])
    gid = jnp.concatenate([batch.astype(jnp.int32),
                           N_GRAPHS + (npad % (G2 - N_GRAPHS))])
    rid2 = rid.reshape(N2 // NCH, 5, 128)
    gid2 = gid.reshape(N2 // NCH, 5, 128)

    W_twoT = W_two.T
    Wenv0T = Wenv0.T
    Wenv1T = Wenv1.T
    Wlat0T = Wlat0.T
    Wlat1T = Wlat1.T
    WoutT = Wout.T

    tsd = _sc_gather_edge_rows(tab, idx2)
    ts = tsd[0].reshape(ER, 128)
    td = tsd[1].reshape(ER, 128)

    grid = (E2 // EB,)
    ebs = pl.BlockSpec((BR, 128), lambda i: (i, 0))
    evs = pl.BlockSpec((SH_DIM, BR, 128), lambda i: (0, i, 0))
    lbs = pl.BlockSpec((LATENT, EB), lambda i: (0, i))
    wspec = lambda s: pl.BlockSpec(s, lambda i: tuple(0 for _ in s))

    ev0 = pl.pallas_call(
        _t2_kernel, grid=grid,
        in_specs=[ebs, ebs, wspec((LATENT, 82)), wspec((MULT, LATENT))],
        out_specs=evs,
        out_shape=jax.ShapeDtypeStruct((SH_DIM, ER, 128), f32),
    )(ts, td, W_twoT, Wenv0T)

    eg0 = _sc_env_round(ev0.reshape(SH_DIM, E2, MULT), dst2)
    eg0 = eg0.reshape(SH_DIM, ER, 128)

    ev1, lat1 = pl.pallas_call(
        _t4_kernel, grid=grid,
        in_specs=[ebs, ebs, evs, wspec((LATENT, 82)),
                  wspec((LATENT, LATENT + MULT)), wspec((MULT, LATENT))],
        out_specs=[evs, lbs],
        out_shape=[jax.ShapeDtypeStruct((SH_DIM, ER, 128), f32),
                   jax.ShapeDtypeStruct((LATENT, E2), f32)],
    )(ts, td, eg0, W_twoT, Wlat0T, Wenv1T)

    eg1 = _sc_env_round(ev1.reshape(SH_DIM, E2, MULT), dst2)
    eg1 = eg1.reshape(SH_DIM, ER, 128)

    ee4 = pl.pallas_call(
        _t6_kernel, grid=grid,
        in_specs=[ebs, ebs, lbs, evs,
                  wspec((LATENT, LATENT + MULT)), wspec((OUT_DIM, LATENT))],
        out_specs=pl.BlockSpec((4, BR, 128), lambda i: (0, i, 0)),
        out_shape=jax.ShapeDtypeStruct((4, ER, 128), f32),
    )(ts, td, lat1, eg1, Wlat1T, WoutT)

    ne = _sc_node_energy(ee4.reshape(4, E2, MULT), dst2)

    rsum, rcnt, gsum, gcnt = _sc_segment_sums(ne, rid2, gid2)

    node_energy = ne[:N]

    residue_embedding, graph_embedding = pl.pallas_call(
        _t10_kernel,
        in_specs=[pl.BlockSpec((R2, OUT_DIM), lambda: (0, 0)),
                  pl.BlockSpec((R2, MULT), lambda: (0, 0)),
                  pl.BlockSpec((G2, OUT_DIM), lambda: (0, 0)),
                  pl.BlockSpec((G2, MULT), lambda: (0, 0))],
        out_specs=[pl.BlockSpec((N_RES, OUT_DIM), lambda: (0, 0)),
                   pl.BlockSpec((N_GRAPHS, OUT_DIM), lambda: (0, 0))],
        out_shape=[jax.ShapeDtypeStruct((N_RES, OUT_DIM), f32),
                   jax.ShapeDtypeStruct((N_GRAPHS, OUT_DIM), f32)],
    )(rsum, rcnt, gsum, gcnt)

    return (node_energy, residue_embedding, graph_embedding)
